# Initial kernel scaffold; baseline (speedup 1.0000x reference)
import jax
import jax.numpy as jnp
from jax.experimental import pallas as pl


def kernel(feature_map, atlas_labels):
    raise NotImplementedError("write your pallas kernel here")


# speedup vs baseline: 1.0800x; 1.0800x over previous
import functools

import jax
import jax.numpy as jnp
from jax import lax
from jax.experimental import pallas as pl
from jax.experimental.pallas import tpu as pltpu
from jax.experimental.pallas import tpu_sc as plsc

NUM_SEG = 201
SEG_PAD = 208
V_TOTAL = 46 * 55 * 46
CHANNELS = 256
NW = 32
CH_PER_W = CHANNELS // NW
CHUNK = 4096
NFULL = V_TOTAL // CHUNK
TAIL = V_TOTAL - NFULL * CHUNK
TAIL_VECS = TAIL // 16
TAIL_REM = TAIL - TAIL_VECS * 16


def _seg_mean_kernel(data_hbm, labels_hbm, out_hbm, cnt_hbm,
                     lab_v, data_v, acc_v, cnt_v):
    cid = lax.axis_index("c")
    sid = lax.axis_index("s")
    wid = sid * 2 + cid
    ch0 = wid * CH_PER_W

    zero16 = jnp.zeros((16,), jnp.float32)
    ones16 = jnp.ones((16,), jnp.float32)

    for r in range(SEG_PAD // 16):
        cnt_v[pl.ds(r * 16, 16)] = zero16
        for ch in range(CH_PER_W):
            acc_v[ch, pl.ds(r * 16, 16)] = zero16

    ch_ids = [jnp.full((16,), ch, jnp.int32) for ch in range(CH_PER_W)]

    def vec_body(j, _):
        base = j * 16
        idx = lab_v[pl.ds(base, 16)]
        plsc.addupdate_scatter(cnt_v, [idx], ones16)
        for ch in range(CH_PER_W):
            vals = data_v[pl.ds(ch * CHUNK + base, 16)]
            plsc.addupdate_scatter(acc_v, [ch_ids[ch], idx], vals)
        return 0

    for k in range(NFULL + 1):
        off = k * CHUNK
        n = CHUNK if k < NFULL else TAIL
        pltpu.sync_copy(labels_hbm.at[pl.ds(off, n)], lab_v.at[pl.ds(0, n)])
        for ch in range(CH_PER_W):
            pltpu.sync_copy(data_hbm.at[ch0 + ch, pl.ds(off, n)],
                            data_v.at[pl.ds(ch * CHUNK, n)])
        lax.fori_loop(0, n // 16, vec_body, 0)
        if n % 16:
            base = (n // 16) * 16
            lane = jnp.arange(16, dtype=jnp.int32)
            mask = lane < (n % 16)
            idx = jnp.minimum(lab_v[pl.ds(base, 16)], NUM_SEG - 1)
            plsc.addupdate_scatter(cnt_v, [idx], ones16, mask=mask)
            for ch in range(CH_PER_W):
                vals = data_v[pl.ds(ch * CHUNK + base, 16)]
                plsc.addupdate_scatter(acc_v, [ch_ids[ch], idx], vals,
                                       mask=mask)

    for r in range(SEG_PAD // 16):
        c = cnt_v[pl.ds(r * 16, 16)]
        nonzero = c > 0.0
        inv_ok = jnp.maximum(c, 1.0)
        for ch in range(CH_PER_W):
            s = acc_v[ch, pl.ds(r * 16, 16)]
            acc_v[ch, pl.ds(r * 16, 16)] = jnp.where(nonzero, s / inv_ok, 0.0)

    pltpu.sync_copy(acc_v, out_hbm.at[pl.ds(ch0, CH_PER_W), :])

    @pl.when(wid == 0)
    def _():
        pltpu.sync_copy(cnt_v, cnt_hbm)


_seg_mean = functools.partial(
    pl.kernel,
    out_type=[
        jax.ShapeDtypeStruct((CHANNELS, SEG_PAD), jnp.float32),
        jax.ShapeDtypeStruct((SEG_PAD,), jnp.float32),
    ],
    mesh=plsc.VectorSubcoreMesh(core_axis_name="c", subcore_axis_name="s"),
    compiler_params=pltpu.CompilerParams(
        needs_layout_passes=False, use_tc_tiling_on_sc=False),
    scratch_types=[
        pltpu.VMEM((CHUNK,), jnp.int32),
        pltpu.VMEM((CH_PER_W * CHUNK,), jnp.float32),
        pltpu.VMEM((CH_PER_W, SEG_PAD), jnp.float32),
        pltpu.VMEM((SEG_PAD,), jnp.float32),
    ],
)(_seg_mean_kernel)


def kernel(feature_map, atlas_labels):
    B, C, D, H, W = feature_map.shape
    data = feature_map.reshape(B * C, D * H * W)
    labels = atlas_labels.reshape(-1).astype(jnp.int32)
    means, counts = _seg_mean(data, labels)
    roi = means[:, 1:NUM_SEG]
    roi_features = roi.reshape(B, C, NUM_SEG - 1).transpose(0, 2, 1)
    valid = counts[1:NUM_SEG] > 0
    roi_valid_mask = jnp.broadcast_to(valid[None, :], (B, NUM_SEG - 1))
    return (roi_features, roi_valid_mask)


# speedup vs baseline: 1.7724x; 1.6411x over previous
import functools

import jax
import jax.numpy as jnp
from jax import lax
from jax.experimental import pallas as pl
from jax.experimental.pallas import tpu as pltpu
from jax.experimental.pallas import tpu_sc as plsc

NUM_SEG = 201
SEG_PAD = 208
V_TOTAL = 46 * 55 * 46
V_MAIN = (V_TOTAL // 128) * 128
V_REST = V_TOTAL - V_MAIN
CHANNELS = 256
NW = 32
CH_PER_W = CHANNELS // NW
CHUNK = 4096
NFULL = V_MAIN // CHUNK
LAST = V_MAIN - NFULL * CHUNK
_CHUNKS = [(k * CHUNK, CHUNK, False) for k in range(NFULL)]
_CHUNKS.append((NFULL * CHUNK, LAST, False))
_CHUNKS.append((V_MAIN, 128, True))
OUT_STRIDE = 256


def _seg_mean_kernel(data_hbm, tail_hbm, labels_hbm, out_hbm, cnt_hbm,
                     lab0, lab1, buf0, buf1,
                     a0, a1, a2, a3, a4, a5, a6, a7, cnt_v,
                     sl0, sd0, sl1, sd1):
    cid = lax.axis_index("c")
    sid = lax.axis_index("s")
    wid = sid * 2 + cid
    ch0 = wid * CH_PER_W

    accs = [a0, a1, a2, a3, a4, a5, a6, a7]
    labs = [lab0, lab1]
    bufs = [buf0, buf1]
    sems = [(sl0, sd0), (sl1, sd1)]

    zero16 = jnp.zeros((16,), jnp.float32)
    ones16 = jnp.ones((16,), jnp.float32)

    for r in range(SEG_PAD // 16):
        cnt_v[pl.ds(r * 16, 16)] = zero16
        for acc in accs:
            acc[pl.ds(r * 16, 16)] = zero16

    def issue(slot, off, n, tail):
        slab, sdat = sems[slot]
        dl = pltpu.async_copy(labels_hbm.at[pl.ds(off, n)],
                              labs[slot].at[pl.ds(0, n)], slab)
        src = tail_hbm if tail else data_hbm
        doff = 0 if tail else off
        dd = pltpu.async_copy(src.at[pl.ds(ch0, CH_PER_W), pl.ds(doff, n)],
                              bufs[slot].at[:, pl.ds(0, n)], sdat)
        return dl, dd

    def process(slot, n):
        lab_v = labs[slot]
        data_v = bufs[slot]

        def vec_body(j, _):
            for u in range(2):
                base = (j * 2 + u) * 16
                idx = lab_v[pl.ds(base, 16)]
                plsc.addupdate_scatter(cnt_v, [idx], ones16)
                for ch in range(CH_PER_W):
                    plsc.addupdate_scatter(accs[ch], [idx],
                                           data_v[ch, pl.ds(base, 16)])
            return 0

        lax.fori_loop(0, n // 32, vec_body, 0)

    pending = issue(0, *_CHUNKS[0])
    for k in range(len(_CHUNKS)):
        nxt = (k + 1) % 2
        if k + 1 < len(_CHUNKS):
            nxt_pending = issue(nxt, *_CHUNKS[k + 1])
        pending[0].wait()
        pending[1].wait()
        process(k % 2, _CHUNKS[k][1])
        if k + 1 < len(_CHUNKS):
            pending = nxt_pending

    for r in range(SEG_PAD // 16):
        c = cnt_v[pl.ds(r * 16, 16)]
        nonzero = c > 0.0
        safe = jnp.maximum(c, 1.0)
        for acc in accs:
            s = acc[pl.ds(r * 16, 16)]
            acc[pl.ds(r * 16, 16)] = jnp.where(nonzero, s / safe, 0.0)
    for ch in range(CH_PER_W):
        pltpu.sync_copy(accs[ch],
                        out_hbm.at[pl.ds((ch0 + ch) * OUT_STRIDE, SEG_PAD)])

    @pl.when(wid == 0)
    def _():
        pltpu.sync_copy(cnt_v, cnt_hbm)


_seg_mean = functools.partial(
    pl.kernel,
    out_type=[
        jax.ShapeDtypeStruct((CHANNELS * OUT_STRIDE,), jnp.float32),
        jax.ShapeDtypeStruct((SEG_PAD,), jnp.float32),
    ],
    mesh=plsc.VectorSubcoreMesh(core_axis_name="c", subcore_axis_name="s"),
    compiler_params=pltpu.CompilerParams(
        needs_layout_passes=False, use_tc_tiling_on_sc=True),
    scratch_types=[
        pltpu.VMEM((CHUNK,), jnp.int32),
        pltpu.VMEM((CHUNK,), jnp.int32),
        pltpu.VMEM((CH_PER_W, CHUNK), jnp.float32),
        pltpu.VMEM((CH_PER_W, CHUNK), jnp.float32),
    ] + [pltpu.VMEM((SEG_PAD,), jnp.float32) for _ in range(CH_PER_W + 1)]
    + [pltpu.SemaphoreType.DMA] * 4,
)(_seg_mean_kernel)


def kernel(feature_map, atlas_labels):
    B, C, D, H, W = feature_map.shape
    data = feature_map.reshape(B * C, D * H * W)
    labels = atlas_labels.reshape(-1).astype(jnp.int32)
    labels_pad = jnp.concatenate(
        [labels, jnp.full((128 - V_REST,), NUM_SEG, jnp.int32)])
    tail = jnp.pad(data[:, V_MAIN:], ((0, 0), (0, 128 - V_REST)))
    flat, counts = _seg_mean(data, tail, labels_pad)
    means = flat.reshape(CHANNELS, OUT_STRIDE)
    roi = means[:, 1:NUM_SEG]
    roi_features = roi.reshape(B, C, NUM_SEG - 1).transpose(0, 2, 1)
    valid = counts[1:NUM_SEG] > 0
    roi_valid_mask = jnp.broadcast_to(valid[None, :], (B, NUM_SEG - 1))
    return (roi_features, roi_valid_mask)


# speedup vs baseline: 2.3533x; 1.3277x over previous
import functools

import jax
import jax.numpy as jnp
from jax import lax
from jax.experimental import pallas as pl
from jax.experimental.pallas import tpu as pltpu
from jax.experimental.pallas import tpu_sc as plsc

NUM_SEG = 201
SEG_PAD = 208
V_TOTAL = 46 * 55 * 46
CHANNELS = 256
NW = 32
BLK = 128
NBLK_FULL = V_TOTAL // BLK
TAIL_ROWS = V_TOTAL - NBLK_FULL * BLK
ACC_WORDS = SEG_PAD * CHANNELS
KMAX = 29


def _seg_sum_kernel(data_hbm, labels_hbm, out_hbm, cnt_hbm,
                    lab0, lab1, buf0, buf1, acc_v, cnt_v,
                    sl0, sd0, sl1, sd1):
    cid = lax.axis_index("c")
    sid = lax.axis_index("s")
    wid = sid * 2 + cid

    labs = [lab0, lab1]
    bufs = [buf0, buf1]
    sems = [(sl0, sd0), (sl1, sd1)]

    zero16 = jnp.zeros((16,), jnp.float32)
    ones16 = jnp.ones((16,), jnp.float32)
    cvecs = [jnp.arange(16, dtype=jnp.int32) + g * 16 for g in range(16)]

    def zbody(z, _):
        acc_v[pl.ds(z * 16, 16)] = zero16
        return 0
    lax.fori_loop(0, ACC_WORDS // 16, zbody, 0)
    for r in range(SEG_PAD // 16):
        cnt_v[pl.ds(r * 16, 16)] = zero16

    def issue(slot, g, nrows):
        slab, sdat = sems[slot]
        pltpu.async_copy(labels_hbm.at[pl.ds(g * BLK, nrows)],
                         labs[slot].at[pl.ds(0, nrows)], slab)
        pltpu.async_copy(data_hbm.at[pl.ds(g * BLK, nrows), :],
                         bufs[slot].at[pl.ds(0, nrows), :], sdat)

    def drain(slot, nrows):
        slab, sdat = sems[slot]
        pltpu.make_async_copy(labels_hbm.at[pl.ds(0, nrows)],
                              labs[slot].at[pl.ds(0, nrows)], slab).wait()
        pltpu.make_async_copy(data_hbm.at[pl.ds(0, nrows), :],
                              bufs[slot].at[pl.ds(0, nrows), :], sdat).wait()

    def process(slot, nrows):
        lab_v = labs[slot]
        buf = bufs[slot]
        for j in range(nrows // 16):
            idx = lab_v[pl.ds(j * 16, 16)]
            plsc.addupdate_scatter(cnt_v, [idx], ones16)
        if nrows % 16:
            base = (nrows // 16) * 16
            mask = jnp.arange(16, dtype=jnp.int32) < (nrows % 16)
            idx = jnp.minimum(lab_v[pl.ds(base, 16)], NUM_SEG - 1)
            plsc.addupdate_scatter(cnt_v, [idx], ones16, mask=mask)

        def vox(v, _):
            vsp = jnp.full((16,), v, jnp.int32)
            lbase = plsc.load_gather(lab_v, [vsp]) << 8
            for g in range(16):
                vals = buf[v, pl.ds(g * 16, 16)]
                plsc.addupdate_scatter(acc_v, [lbase + cvecs[g]], vals)
            return 0
        lax.fori_loop(0, nrows, vox, 0)

    issue(0, wid, BLK)

    def ring(i, _):
        k1 = 2 * i + 1
        issue(1, k1 * NW + wid, BLK)
        drain(0, BLK)
        process(0, BLK)

        @pl.when(i < (KMAX - 1) // 2 - 1)
        def _():
            issue(0, (2 * i + 2) * NW + wid, BLK)
        drain(1, BLK)
        process(1, BLK)
        return 0
    lax.fori_loop(0, (KMAX - 1) // 2, ring, 0)

    g_last = (KMAX - 1) * NW + wid

    @pl.when(wid < 13)
    def _():
        issue(0, g_last, BLK)
        drain(0, BLK)
        process(0, BLK)

    @pl.when(wid == 13)
    def _():
        issue(0, g_last, TAIL_ROWS)
        drain(0, TAIL_ROWS)
        process(0, TAIL_ROWS)

    pltpu.sync_copy(acc_v, out_hbm.at[pl.ds(wid * ACC_WORDS, ACC_WORDS)])
    pltpu.sync_copy(cnt_v, cnt_hbm.at[pl.ds(wid * SEG_PAD, SEG_PAD)])


_seg_sum = functools.partial(
    pl.kernel,
    out_type=[
        jax.ShapeDtypeStruct((NW * ACC_WORDS,), jnp.float32),
        jax.ShapeDtypeStruct((NW * SEG_PAD,), jnp.float32),
    ],
    mesh=plsc.VectorSubcoreMesh(core_axis_name="c", subcore_axis_name="s"),
    compiler_params=pltpu.CompilerParams(
        needs_layout_passes=False, use_tc_tiling_on_sc=False),
    scratch_types=[
        pltpu.VMEM((BLK,), jnp.int32),
        pltpu.VMEM((BLK,), jnp.int32),
        pltpu.VMEM((BLK, CHANNELS), jnp.float32),
        pltpu.VMEM((BLK, CHANNELS), jnp.float32),
        pltpu.VMEM((ACC_WORDS,), jnp.float32),
        pltpu.VMEM((SEG_PAD,), jnp.float32),
    ] + [pltpu.SemaphoreType.DMA] * 4,
)(_seg_sum_kernel)


def kernel(feature_map, atlas_labels):
    B, C, D, H, W = feature_map.shape
    V = D * H * W
    dataT = feature_map.transpose(2, 3, 4, 0, 1).reshape(V, B * C)
    labels = atlas_labels.reshape(-1).astype(jnp.int32)
    parts, pcnts = _seg_sum(dataT, labels)
    sums = parts.reshape(NW, SEG_PAD, CHANNELS).sum(0)
    counts = pcnts.reshape(NW, SEG_PAD).sum(0)
    means = jnp.where(counts[:, None] > 0,
                      sums / jnp.maximum(counts[:, None], 1.0), 0.0)
    roi = means[1:NUM_SEG]
    roi_features = roi.reshape(NUM_SEG - 1, B, C).transpose(1, 0, 2)
    valid = counts[1:NUM_SEG] > 0
    roi_valid_mask = jnp.broadcast_to(valid[None, :], (B, NUM_SEG - 1))
    return (roi_features, roi_valid_mask)


# speedup vs baseline: 3.8234x; 1.6247x over previous
import functools

import jax
import jax.numpy as jnp
from jax import lax
from jax.experimental import pallas as pl
from jax.experimental.pallas import tpu as pltpu
from jax.experimental.pallas import tpu_sc as plsc

NUM_SEG = 201
SEG_PAD = 208
V_TOTAL = 46 * 55 * 46
CHANNELS = 256
NW = 32
NS = 16
BLK = 128
NBLK_FULL = V_TOTAL // BLK
TAIL_ROWS = V_TOTAL - NBLK_FULL * BLK
KMAX = 29
ZROWS = SEG_PAD // NS


def _seg_sum_kernel(data_hbm, labels_hbm, out_a, out_b, cnt_hbm,
                    lab0, lab1, buf0, buf1, lab_t, buf_t,
                    zbuf, cnt_v, table,
                    sl0, sd0, sl1, sd1, st):
    cid = lax.axis_index("c")
    sid = lax.axis_index("s")
    wid = sid * 2 + cid

    labs = [lab0, lab1]
    bufs = [buf0, buf1]
    sems = [(sl0, sd0), (sl1, sd1)]

    zero16 = jnp.zeros((16,), jnp.float32)
    ones16 = jnp.ones((16,), jnp.float32)

    for r in range(SEG_PAD // 16):
        cnt_v[pl.ds(r * 16, 16)] = zero16
    for r in range(ZROWS):
        for g in range(CHANNELS // 16):
            zbuf[r, pl.ds(g * 16, 16)] = zero16
    pltpu.sync_copy(zbuf, table.at[pl.ds(sid * ZROWS, ZROWS), :])
    plsc.subcore_barrier()

    def issue(slot, g):
        slab, sdat = sems[slot]
        pltpu.async_copy(labels_hbm.at[pl.ds(g * BLK, BLK)], labs[slot], slab)
        pltpu.async_copy(data_hbm.at[pl.ds(g * BLK, BLK), :], bufs[slot], sdat)

    def drain(slot):
        slab, sdat = sems[slot]
        pltpu.make_async_copy(labels_hbm.at[pl.ds(0, BLK)],
                              labs[slot], slab).wait()
        pltpu.make_async_copy(data_hbm.at[pl.ds(0, BLK), :],
                              bufs[slot], sdat).wait()

    def process(slot):
        lab_v = labs[slot]
        for j in range(BLK // 16):
            idx = lab_v[pl.ds(j * 16, 16)]
            plsc.addupdate_scatter(cnt_v, [idx], ones16)
        pltpu.sync_copy(bufs[slot], table.at[lab_v], add=True)

    issue(0, wid)

    def ring(i, _):
        issue(1, (2 * i + 1) * NW + wid)
        drain(0)
        process(0)

        @pl.when(i < (KMAX - 1) // 2 - 1)
        def _():
            issue(0, (2 * i + 2) * NW + wid)
        drain(1)
        process(1)
        return 0
    lax.fori_loop(0, (KMAX - 1) // 2, ring, 0)

    g_last = (KMAX - 1) * NW + wid

    @pl.when(wid < 13)
    def _():
        issue(0, g_last)
        drain(0)
        process(0)

    @pl.when(wid == 13)
    def _():
        pltpu.async_copy(labels_hbm.at[pl.ds(g_last * BLK, TAIL_ROWS)],
                         lab_t, st)
        pltpu.make_async_copy(labels_hbm.at[pl.ds(0, TAIL_ROWS)],
                              lab_t, st).wait()
        pltpu.async_copy(data_hbm.at[pl.ds(g_last * BLK, TAIL_ROWS), :],
                         buf_t, st)
        pltpu.make_async_copy(data_hbm.at[pl.ds(0, TAIL_ROWS), :],
                              buf_t, st).wait()
        idx = lab_t[pl.ds(0, 16)]
        plsc.addupdate_scatter(cnt_v, [idx], ones16)
        mask = jnp.arange(16, dtype=jnp.int32) >= 4
        idx2 = lab_t[pl.ds(TAIL_ROWS - 16, 16)]
        plsc.addupdate_scatter(cnt_v, [idx2], ones16, mask=mask)
        pltpu.sync_copy(buf_t, table.at[lab_t], add=True)

    plsc.subcore_barrier()
    pltpu.sync_copy(cnt_v, cnt_hbm.at[pl.ds(wid * SEG_PAD, SEG_PAD)])

    @pl.when((sid == 0) & (cid == 0))
    def _():
        pltpu.sync_copy(table, out_a)

    @pl.when((sid == 0) & (cid == 1))
    def _():
        pltpu.sync_copy(table, out_b)


_seg_sum = functools.partial(
    pl.kernel,
    out_type=[
        jax.ShapeDtypeStruct((SEG_PAD, CHANNELS), jnp.float32),
        jax.ShapeDtypeStruct((SEG_PAD, CHANNELS), jnp.float32),
        jax.ShapeDtypeStruct((NW * SEG_PAD,), jnp.float32),
    ],
    mesh=plsc.VectorSubcoreMesh(core_axis_name="c", subcore_axis_name="s"),
    compiler_params=pltpu.CompilerParams(
        needs_layout_passes=False, use_tc_tiling_on_sc=False),
    scratch_types=[
        pltpu.VMEM((BLK,), jnp.int32),
        pltpu.VMEM((BLK,), jnp.int32),
        pltpu.VMEM((BLK, CHANNELS), jnp.float32),
        pltpu.VMEM((BLK, CHANNELS), jnp.float32),
        pltpu.VMEM((TAIL_ROWS,), jnp.int32),
        pltpu.VMEM((TAIL_ROWS, CHANNELS), jnp.float32),
        pltpu.VMEM((ZROWS, CHANNELS), jnp.float32),
        pltpu.VMEM((SEG_PAD,), jnp.float32),
        pltpu.VMEM_SHARED((SEG_PAD, CHANNELS), jnp.float32),
    ] + [pltpu.SemaphoreType.DMA] * 5,
)(_seg_sum_kernel)


def kernel(feature_map, atlas_labels):
    B, C, D, H, W = feature_map.shape
    V = D * H * W
    dataT = feature_map.transpose(2, 3, 4, 0, 1).reshape(V, B * C)
    labels = atlas_labels.reshape(-1).astype(jnp.int32)
    tab_a, tab_b, pcnts = _seg_sum(dataT, labels)
    sums = tab_a + tab_b
    counts = pcnts.reshape(NW, SEG_PAD).sum(0)
    means = jnp.where(counts[:, None] > 0,
                      sums / jnp.maximum(counts[:, None], 1.0), 0.0)
    roi = means[1:NUM_SEG]
    roi_features = roi.reshape(NUM_SEG - 1, B, C).transpose(1, 0, 2)
    valid = counts[1:NUM_SEG] > 0
    roi_valid_mask = jnp.broadcast_to(valid[None, :], (B, NUM_SEG - 1))
    return (roi_features, roi_valid_mask)


# speedup vs baseline: 4.1853x; 1.0947x over previous
import functools

import jax
import jax.numpy as jnp
from jax import lax
from jax.experimental import pallas as pl
from jax.experimental.pallas import tpu as pltpu
from jax.experimental.pallas import tpu_sc as plsc

NUM_SEG = 201
SEG_PAD = 208
V_TOTAL = 46 * 55 * 46
CHANNELS = 256
NB, NCH = 4, 64
NW = 32
BLK = 32
NBLK_FULL = V_TOTAL // BLK
TAIL_ROWS = V_TOTAL - NBLK_FULL * BLK
KFULL = 113
REMW = NBLK_FULL - KFULL * NW
ACC_WORDS = SEG_PAD * CHANNELS

_GDN = lax.GatherDimensionNumbers(
    offset_dims=(), collapsed_slice_dims=(0,), start_index_map=(0,))


def _splat(vec, u):
    return lax.gather(vec, jnp.full((16, 1), u, jnp.int32), _GDN, (1,),
                      mode=lax.GatherScatterMode.PROMISE_IN_BOUNDS)


def _seg_sum_kernel(data_hbm, labels_hbm, out_hbm, cnt_hbm,
                    lab0, lab1, buf0, buf1, lab_t, buf_t, acc_v, cnt_v,
                    sl0, sd0, sl1, sd1, st):
    cid = lax.axis_index("c")
    sid = lax.axis_index("s")
    wid = sid * 2 + cid

    labs = [lab0, lab1]
    bufs = [buf0, buf1]
    sems = [(sl0, sd0), (sl1, sd1)]

    zero16 = jnp.zeros((16,), jnp.float32)
    ones16 = jnp.ones((16,), jnp.float32)
    cvecs = [jnp.arange(16, dtype=jnp.int32) + g * 16 for g in range(16)]

    def zbody(z, _):
        acc_v[pl.ds(z * 16, 16)] = zero16
        return 0
    lax.fori_loop(0, ACC_WORDS // 16, zbody, 0)
    for r in range(SEG_PAD // 16):
        cnt_v[pl.ds(r * 16, 16)] = zero16

    def issue(slot, g):
        slab, sdat = sems[slot]
        pltpu.async_copy(labels_hbm.at[pl.ds(g * BLK, BLK)], labs[slot], slab)
        pltpu.async_copy(data_hbm.at[pl.ds(g * BLK, BLK), :, :], bufs[slot],
                         sdat)

    def drain(slot):
        slab, sdat = sems[slot]
        pltpu.make_async_copy(labels_hbm.at[pl.ds(0, BLK)],
                              labs[slot], slab).wait()
        pltpu.make_async_copy(data_hbm.at[pl.ds(0, BLK), :, :],
                              bufs[slot], sdat).wait()

    def process(slot):
        lab_v = labs[slot]
        buf = bufs[slot]

        def vgroup(j, _):
            lv = lab_v[pl.ds(j * 16, 16)]
            plsc.addupdate_scatter(cnt_v, [lv], ones16)
            for u in range(16):
                v = j * 16 + u
                lbase = _splat(lv, u) << 8
                for b in range(NB):
                    for g in range(NCH // 16):
                        vals = buf[v, b, pl.ds(g * 16, 16)]
                        plsc.addupdate_scatter(
                            acc_v, [lbase + cvecs[b * 4 + g]], vals)
            return 0
        lax.fori_loop(0, BLK // 16, vgroup, 0)

    issue(0, wid)

    def ring(i, _):
        issue(1, (2 * i + 1) * NW + wid)
        drain(0)
        process(0)
        issue(0, (2 * i + 2) * NW + wid)
        drain(1)
        process(1)
        return 0
    lax.fori_loop(0, (KFULL - 1) // 2, ring, 0)

    drain(0)
    process(0)

    @pl.when(wid < REMW)
    def _():
        issue(0, KFULL * NW + wid)
        drain(0)
        process(0)

    @pl.when(wid == REMW)
    def _():
        off = NBLK_FULL * BLK
        pltpu.async_copy(labels_hbm.at[pl.ds(off, TAIL_ROWS)], lab_t, st)
        pltpu.make_async_copy(labels_hbm.at[pl.ds(0, TAIL_ROWS)],
                              lab_t, st).wait()
        pltpu.async_copy(data_hbm.at[pl.ds(off, TAIL_ROWS), :, :], buf_t, st)
        pltpu.make_async_copy(data_hbm.at[pl.ds(0, TAIL_ROWS), :, :],
                              buf_t, st).wait()
        for j in range(2):
            base = j * 12
            lv = lab_t[pl.ds(base, 16)]
            mask = jnp.arange(16, dtype=jnp.int32) >= (4 * j)
            plsc.addupdate_scatter(cnt_v, [lv], ones16, mask=mask)
        def tvox(v, _):
            vsp = jnp.full((16,), v, jnp.int32)
            lbase = plsc.load_gather(lab_t, [vsp]) << 8
            for b in range(NB):
                for g in range(NCH // 16):
                    vals = buf_t[v, b, pl.ds(g * 16, 16)]
                    plsc.addupdate_scatter(
                        acc_v, [lbase + cvecs[b * 4 + g]], vals)
            return 0
        lax.fori_loop(0, TAIL_ROWS, tvox, 0)

    pltpu.sync_copy(acc_v, out_hbm.at[pl.ds(wid * ACC_WORDS, ACC_WORDS)])
    pltpu.sync_copy(cnt_v, cnt_hbm.at[pl.ds(wid * SEG_PAD, SEG_PAD)])


_seg_sum = functools.partial(
    pl.kernel,
    out_type=[
        jax.ShapeDtypeStruct((NW * ACC_WORDS,), jnp.float32),
        jax.ShapeDtypeStruct((NW * SEG_PAD,), jnp.float32),
    ],
    mesh=plsc.VectorSubcoreMesh(core_axis_name="c", subcore_axis_name="s"),
    compiler_params=pltpu.CompilerParams(
        needs_layout_passes=False, use_tc_tiling_on_sc=True),
    scratch_types=[
        pltpu.VMEM((BLK,), jnp.int32),
        pltpu.VMEM((BLK,), jnp.int32),
        pltpu.VMEM((BLK, NB, NCH), jnp.float32),
        pltpu.VMEM((BLK, NB, NCH), jnp.float32),
        pltpu.VMEM((TAIL_ROWS,), jnp.int32),
        pltpu.VMEM((TAIL_ROWS, NB, NCH), jnp.float32),
        pltpu.VMEM((ACC_WORDS,), jnp.float32),
        pltpu.VMEM((SEG_PAD,), jnp.float32),
    ] + [pltpu.SemaphoreType.DMA] * 5,
)(_seg_sum_kernel)


def kernel(feature_map, atlas_labels):
    B, C, D, H, W = feature_map.shape
    V = D * H * W
    dataT = feature_map.transpose(2, 3, 4, 0, 1).reshape(V, B, C)
    labels = atlas_labels.reshape(-1).astype(jnp.int32)
    parts, pcnts = _seg_sum(dataT, labels)
    sums = parts.reshape(NW, SEG_PAD, NB, NCH).sum(0)
    counts = pcnts.reshape(NW, SEG_PAD).sum(0)
    cn = counts[:, None, None]
    means = jnp.where(cn > 0, sums / jnp.maximum(cn, 1.0), 0.0)
    roi = means[1:NUM_SEG]
    roi_features = roi.transpose(1, 0, 2)
    valid = counts[1:NUM_SEG] > 0
    roi_valid_mask = jnp.broadcast_to(valid[None, :], (B, NUM_SEG - 1))
    return (roi_features, roi_valid_mask)
